# Initial kernel scaffold; baseline (speedup 1.0000x reference)
import jax
import jax.numpy as jnp
from jax.experimental import pallas as pl


def kernel(p_start, p_end, state_start, state_end, state_target, params):
    raise NotImplementedError("write your pallas kernel here")


# speedup vs baseline: 2.5828x; 2.5828x over previous
import functools

import jax
import jax.numpy as jnp
from jax import lax
from jax.experimental import pallas as pl
from jax.experimental.pallas import tpu as pltpu
from jax.experimental.pallas import tpu_sc as plsc

B = 4
T = 2048
HID = 128
TWOH = 256
RESO = 64
NCELL = RESO * RESO
NT = 4
TT = T // NT
FPT = 16
NEG = -3.0e38
F32 = jnp.float32


def _attn_body(states, psT, peT, sw1, sb1, sw2, sb2, sw3, sb3,
               wsT, bs, weT, be, w0T, b0, w1T, b1, wscT,
               netT_out, idx_out, sf_out):
    sv = states[...]
    outs = []
    for i in range(3):
        s = sv[:, i:i + 1]
        h = jnp.maximum(s * sw1[i] + sb1[i], 0.0)
        h = jnp.maximum(
            jnp.dot(h, sw2[i], preferred_element_type=F32) + sb2[i], 0.0)
        outs.append(jnp.dot(h, sw3[i], preferred_element_type=F32) + sb3[i])
    ss, se, st = outs
    st = (ss + st) * 0.5
    sf_out[...] = jnp.concatenate([ss, se, st], axis=1)

    ps = psT[0]
    pe = peT[0]
    fsT = jnp.dot(wsT[...], ps, preferred_element_type=F32) + bs[...]
    feT = jnp.dot(weT[...], pe, preferred_element_type=F32) + be[...]
    fsT = fsT * (1.0 / 16.0)
    scores = lax.dot_general(fsT, feT, (((0,), (0,)), ((), ())),
                             preferred_element_type=F32)
    m = jnp.max(scores, axis=1, keepdims=True)
    e = jnp.exp(scores - m)
    attn = e / jnp.sum(e, axis=1, keepdims=True)
    n0T = lax.dot_general(feT, attn, (((1,), (1,)), ((), ())),
                          preferred_element_type=F32)

    r = jnp.maximum(n0T, 0.0)
    h = jnp.maximum(jnp.dot(w0T[...], r, preferred_element_type=F32) + b0[...], 0.0)
    dx = jnp.dot(w1T[...], h, preferred_element_type=F32) + b1[...]
    netT_out[0] = jnp.dot(wscT[...], n0T, preferred_element_type=F32) + dx

    u0 = jnp.clip(ps[0:1, :] / 1.001 + 0.5, 0.0, 1.0 - 1e-6)
    u2 = jnp.clip(ps[2:3, :] / 1.001 + 0.5, 0.0, 1.0 - 1e-6)
    xi = (u0 * RESO).astype(jnp.int32)
    zi = (u2 * RESO).astype(jnp.int32)
    idx_out[0] = xi + RESO * zi


def _run_attn(states, psT, peT, sws, wsT, bs, weT, be, blk0T):
    w0T, b0, w1T, b1, wscT = blk0T
    sw1, sb1, sw2, sb2, sw3, sb3 = sws
    cst = lambda *dims: (lambda b, j: tuple(0 for _ in dims))
    return pl.pallas_call(
        _attn_body,
        grid=(B, NT),
        in_specs=[
            pl.BlockSpec((B, 3), lambda b, j: (0, 0)),
            pl.BlockSpec((1, 4, TT), lambda b, j: (b, 0, j)),
            pl.BlockSpec((1, 4, T), lambda b, j: (b, 0, 0)),
            pl.BlockSpec(sw1.shape, lambda b, j: (0, 0, 0)),
            pl.BlockSpec(sb1.shape, lambda b, j: (0, 0, 0)),
            pl.BlockSpec(sw2.shape, lambda b, j: (0, 0, 0)),
            pl.BlockSpec(sb2.shape, lambda b, j: (0, 0, 0)),
            pl.BlockSpec(sw3.shape, lambda b, j: (0, 0, 0)),
            pl.BlockSpec(sb3.shape, lambda b, j: (0, 0, 0)),
            pl.BlockSpec(wsT.shape, lambda b, j: (0, 0)),
            pl.BlockSpec(bs.shape, lambda b, j: (0, 0)),
            pl.BlockSpec(weT.shape, lambda b, j: (0, 0)),
            pl.BlockSpec(be.shape, lambda b, j: (0, 0)),
            pl.BlockSpec(w0T.shape, lambda b, j: (0, 0)),
            pl.BlockSpec(b0.shape, lambda b, j: (0, 0)),
            pl.BlockSpec(w1T.shape, lambda b, j: (0, 0)),
            pl.BlockSpec(b1.shape, lambda b, j: (0, 0)),
            pl.BlockSpec(wscT.shape, lambda b, j: (0, 0)),
        ],
        out_specs=[
            pl.BlockSpec((1, HID, TT), lambda b, j: (b, 0, j)),
            pl.BlockSpec((1, 1, TT), lambda b, j: (b, 0, j)),
            pl.BlockSpec((B, 3 * 256), lambda b, j: (0, 0)),
        ],
        out_shape=[
            jax.ShapeDtypeStruct((B, HID, T), F32),
            jax.ShapeDtypeStruct((B, 1, T), jnp.int32),
            jax.ShapeDtypeStruct((B, 3 * 256), F32),
        ],
    )(states, psT, peT, sw1, sb1, sw2, sb2, sw3, sb3,
      wsT, bs, weT, be, w0T, b0, w1T, b1, wscT)


def _sc_pool_body(idx_hbm, netT_hbm, out_hbm, idx_v, feat_v, table,
                  k_scr, v_scr):
    cid = lax.axis_index("c")
    sid = lax.axis_index("s")
    wid = sid * 2 + cid
    b = wid // 8
    f0 = (wid % 8) * FPT

    pltpu.sync_copy(idx_hbm.at[b, 0], idx_v)
    pltpu.sync_copy(netT_hbm.at[b, pl.ds(f0, FPT), :], feat_v)

    lanes = lax.iota(jnp.int32, 16)
    shift_idx = [jnp.maximum(lanes - s, 0) for s in (1, 2, 4, 8)]
    ge_masks = [lanes >= s for s in (1, 2, 4, 8)]
    succ = jnp.minimum(lanes + 1, 15)
    last = lanes == 15
    neg = jnp.full((16,), NEG, F32)

    def init_body(i, carry):
        table[pl.ds(i * 16, 16)] = neg
        return carry
    lax.fori_loop(0, (FPT * NCELL) // 16, init_body, 0, unroll=8)

    def group_body(g, carry):
        base = g * 16
        c = idx_v[pl.ds(base, 16)]
        sk, p = plsc.sort_key_val(c, lanes)
        k_scr[...] = sk
        masks = []
        for si, ge in zip(shift_idx, ge_masks):
            pk = plsc.load_gather(k_scr, [si])
            masks.append((sk == pk) & ge)
        nk = plsc.load_gather(k_scr, [succ])
        endm = (sk != nk) | last
        for f in range(FPT):
            v = feat_v[f, pl.ds(base, 16)]
            v_scr[...] = v
            vs = plsc.load_gather(v_scr, [p])
            for m, si in zip(masks, shift_idx):
                v_scr[...] = vs
                sv = plsc.load_gather(v_scr, [si])
                vs = jnp.where(m, jnp.maximum(vs, sv), vs)
            off = sk + f * NCELL
            old = plsc.load_gather(table, [off])
            plsc.store_scatter(table, [off], jnp.maximum(old, vs), mask=endm)
        return carry
    lax.fori_loop(0, T // 16, group_body, 0)

    def back_body(g, carry):
        base = g * 16
        c = idx_v[pl.ds(base, 16)]
        for f in range(FPT):
            feat_v[f, pl.ds(base, 16)] = plsc.load_gather(table, [c + f * NCELL])
        return carry
    lax.fori_loop(0, T // 16, back_body, 0)

    pltpu.sync_copy(feat_v, out_hbm.at[b, pl.ds(f0, FPT), :])


_sc_pool = pl.kernel(
    _sc_pool_body,
    out_type=jax.ShapeDtypeStruct((B, HID, T), F32),
    mesh=plsc.VectorSubcoreMesh(core_axis_name="c", subcore_axis_name="s"),
    compiler_params=pltpu.CompilerParams(needs_layout_passes=False),
    scratch_types=[
        pltpu.VMEM((T,), jnp.int32),
        pltpu.VMEM((FPT, T), F32),
        pltpu.VMEM((FPT * NCELL,), F32),
        pltpu.VMEM((16,), jnp.int32),
        pltpu.VMEM((16,), F32),
    ],
)


def _block_math(netT, poolT, w0aT, w0bT, b0, w1T, b1, wscaT, wscbT):
    x = netT[0]
    p = poolT[0]
    rx = jnp.maximum(x, 0.0)
    rp = jnp.maximum(p, 0.0)
    h = (jnp.dot(w0aT[...], rx, preferred_element_type=F32)
         + jnp.dot(w0bT[...], rp, preferred_element_type=F32) + b0[...])
    h = jnp.maximum(h, 0.0)
    dx = jnp.dot(w1T[...], h, preferred_element_type=F32) + b1[...]
    return (jnp.dot(wscaT[...], x, preferred_element_type=F32)
            + jnp.dot(wscbT[...], p, preferred_element_type=F32) + dx)


def _block_body(netT, poolT, w0aT, w0bT, b0, w1T, b1, wscaT, wscbT, out):
    out[0] = _block_math(netT, poolT, w0aT, w0bT, b0, w1T, b1, wscaT, wscbT)


def _final_body(netT, poolT, w0aT, w0bT, b0, w1T, b1, wscaT, wscbT,
                fcw, fcb, c_out):
    o = _block_math(netT, poolT, w0aT, w0bT, b0, w1T, b1, wscaT, wscbT)
    c_out[0] = lax.dot_general(o, fcw[...], (((0,), (0,)), ((), ())),
                               preferred_element_type=F32) + fcb[...]


def _wspec(a):
    return pl.BlockSpec(a.shape, lambda b: tuple(0 for _ in a.shape))


def _run_block(netT, poolT, wts):
    specs = ([pl.BlockSpec((1, HID, T), lambda b: (b, 0, 0))] * 2
             + [_wspec(w) for w in wts])
    return pl.pallas_call(
        _block_body,
        grid=(B,),
        in_specs=specs,
        out_specs=pl.BlockSpec((1, HID, T), lambda b: (b, 0, 0)),
        out_shape=jax.ShapeDtypeStruct((B, HID, T), F32),
    )(netT, poolT, *wts)


def _run_final(netT, poolT, wts):
    specs = ([pl.BlockSpec((1, HID, T), lambda b: (b, 0, 0))] * 2
             + [_wspec(w) for w in wts])
    return pl.pallas_call(
        _final_body,
        grid=(B,),
        in_specs=specs,
        out_specs=pl.BlockSpec((1, T, 64), lambda b: (b, 0, 0)),
        out_shape=jax.ShapeDtypeStruct((B, T, 64), F32),
    )(netT, poolT, *wts)


def kernel(p_start, p_end, state_start, state_end, state_target, params):
    prm = params
    psT = jnp.swapaxes(p_start, 1, 2)
    peT = jnp.swapaxes(p_end, 1, 2)
    states = jnp.stack([state_start, state_end, state_target], axis=1)

    sw1 = jnp.stack([prm['lin1'][0], prm['lin4'][0], prm['lin7'][0]])
    sb1 = jnp.stack([prm['lin1'][1], prm['lin4'][1], prm['lin7'][1]])[:, None, :]
    sw2 = jnp.stack([prm['lin2'][0], prm['lin5'][0], prm['lin8'][0]])
    sb2 = jnp.stack([prm['lin2'][1], prm['lin5'][1], prm['lin8'][1]])[:, None, :]
    sw3 = jnp.stack([prm['lin3'][0], prm['lin6'][0], prm['lin9'][0]])
    sb3 = jnp.stack([prm['lin3'][1], prm['lin6'][1], prm['lin9'][1]])[:, None, :]

    wsT = prm['Ws'][0].T
    bs = prm['Ws'][1][:, None]
    weT = prm['We'][0].T
    be = prm['We'][1][:, None]

    w0, b0, w1, b1, wsc = prm['blocks'][0]
    blk0T = (w0.T, b0[:, None], w1.T, b1[:, None], wsc.T)

    netT, idx3, state_feat = _run_attn(
        states, psT, peT, (sw1, sb1, sw2, sb2, sw3, sb3),
        wsT, bs, weT, be, blk0T)

    def blk_wts(blk):
        w0, b0, w1, b1, wsc = blk
        return (w0[:HID].T, w0[HID:].T, b0[:, None], w1.T, b1[:, None],
                wsc[:HID].T, wsc[HID:].T)

    for blk in prm['blocks'][1:-1]:
        poolT = _sc_pool(idx3, netT)
        netT = _run_block(netT, poolT, blk_wts(blk))

    poolT = _sc_pool(idx3, netT)
    fcw, fcb = prm['fc_c']
    c = _run_final(netT, poolT, blk_wts(prm['blocks'][-1]) + (fcw, fcb[None, :]))
    return (c, state_feat)


# speedup vs baseline: 3.2403x; 1.2546x over previous
import functools

import jax
import jax.numpy as jnp
from jax import lax
from jax.experimental import pallas as pl
from jax.experimental.pallas import tpu as pltpu
from jax.experimental.pallas import tpu_sc as plsc

B = 4
T = 2048
HID = 128
TWOH = 256
RESO = 64
NCELL = RESO * RESO
NT = 4
TT = T // NT
FPT = 16
NEG = -3.0e38
F32 = jnp.float32


def _attn_body(states, psT, peT, sw1, sb1, sw2, sb2, sw3, sb3,
               wsT, bs, weT, be, w0T, b0, w1T, b1, wscT,
               netT_out, idx_out, sf_out):
    sv = states[...]
    outs = []
    for i in range(3):
        s = sv[:, i:i + 1]
        h = jnp.maximum(s * sw1[i] + sb1[i], 0.0)
        h = jnp.maximum(
            jnp.dot(h, sw2[i], preferred_element_type=F32) + sb2[i], 0.0)
        outs.append(jnp.dot(h, sw3[i], preferred_element_type=F32) + sb3[i])
    ss, se, st = outs
    st = (ss + st) * 0.5
    sf_out[...] = jnp.concatenate([ss, se, st], axis=1)

    ps = psT[0]
    pe = peT[0]
    fsT = jnp.dot(wsT[...], ps, preferred_element_type=F32) + bs[...]
    feT = jnp.dot(weT[...], pe, preferred_element_type=F32) + be[...]
    fsT = fsT * (1.0 / 16.0)
    scores = lax.dot_general(fsT, feT, (((0,), (0,)), ((), ())),
                             preferred_element_type=F32)
    m = jnp.max(scores, axis=1, keepdims=True)
    e = jnp.exp(scores - m)
    attn = e / jnp.sum(e, axis=1, keepdims=True)
    n0T = lax.dot_general(feT, attn, (((1,), (1,)), ((), ())),
                          preferred_element_type=F32)

    r = jnp.maximum(n0T, 0.0)
    h = jnp.maximum(jnp.dot(w0T[...], r, preferred_element_type=F32) + b0[...], 0.0)
    dx = jnp.dot(w1T[...], h, preferred_element_type=F32) + b1[...]
    netT_out[0] = jnp.dot(wscT[...], n0T, preferred_element_type=F32) + dx

    u0 = jnp.clip(ps[0:1, :] / 1.001 + 0.5, 0.0, 1.0 - 1e-6)
    u2 = jnp.clip(ps[2:3, :] / 1.001 + 0.5, 0.0, 1.0 - 1e-6)
    xi = (u0 * RESO).astype(jnp.int32)
    zi = (u2 * RESO).astype(jnp.int32)
    idx_out[0] = xi + RESO * zi


def _run_attn(states, psT, peT, sws, wsT, bs, weT, be, blk0T):
    w0T, b0, w1T, b1, wscT = blk0T
    sw1, sb1, sw2, sb2, sw3, sb3 = sws
    cst = lambda *dims: (lambda b, j: tuple(0 for _ in dims))
    return pl.pallas_call(
        _attn_body,
        grid=(B, NT),
        in_specs=[
            pl.BlockSpec((B, 3), lambda b, j: (0, 0)),
            pl.BlockSpec((1, 4, TT), lambda b, j: (b, 0, j)),
            pl.BlockSpec((1, 4, T), lambda b, j: (b, 0, 0)),
            pl.BlockSpec(sw1.shape, lambda b, j: (0, 0, 0)),
            pl.BlockSpec(sb1.shape, lambda b, j: (0, 0, 0)),
            pl.BlockSpec(sw2.shape, lambda b, j: (0, 0, 0)),
            pl.BlockSpec(sb2.shape, lambda b, j: (0, 0, 0)),
            pl.BlockSpec(sw3.shape, lambda b, j: (0, 0, 0)),
            pl.BlockSpec(sb3.shape, lambda b, j: (0, 0, 0)),
            pl.BlockSpec(wsT.shape, lambda b, j: (0, 0)),
            pl.BlockSpec(bs.shape, lambda b, j: (0, 0)),
            pl.BlockSpec(weT.shape, lambda b, j: (0, 0)),
            pl.BlockSpec(be.shape, lambda b, j: (0, 0)),
            pl.BlockSpec(w0T.shape, lambda b, j: (0, 0)),
            pl.BlockSpec(b0.shape, lambda b, j: (0, 0)),
            pl.BlockSpec(w1T.shape, lambda b, j: (0, 0)),
            pl.BlockSpec(b1.shape, lambda b, j: (0, 0)),
            pl.BlockSpec(wscT.shape, lambda b, j: (0, 0)),
        ],
        out_specs=[
            pl.BlockSpec((1, HID, TT), lambda b, j: (b, 0, j)),
            pl.BlockSpec((1, 1, TT), lambda b, j: (b, 0, j)),
            pl.BlockSpec((B, 3 * 256), lambda b, j: (0, 0)),
        ],
        out_shape=[
            jax.ShapeDtypeStruct((B, HID, T), F32),
            jax.ShapeDtypeStruct((B, 1, T), jnp.int32),
            jax.ShapeDtypeStruct((B, 3 * 256), F32),
        ],
    )(states, psT, peT, sw1, sb1, sw2, sb2, sw3, sb3,
      wsT, bs, weT, be, w0T, b0, w1T, b1, wscT)


def _vshuf(v, i):
    return jnp.take_along_axis(v, i, axis=0)


def _sc_pool_body(idx_hbm, netT_hbm, out_hbm, idx_v, feat_v, *tables):
    cid = lax.axis_index("c")
    sid = lax.axis_index("s")
    wid = sid * 2 + cid
    b = wid // 8
    f0 = (wid % 8) * FPT

    pltpu.sync_copy(idx_hbm.at[b, 0], idx_v)
    pltpu.sync_copy(netT_hbm.at[b, pl.ds(f0, FPT), :], feat_v)

    lanes = lax.iota(jnp.int32, 16)
    shift_idx = [jnp.maximum(lanes - s, 0) for s in (1, 2, 4, 8)]
    ge_masks = [lanes >= s for s in (1, 2, 4, 8)]
    succ = jnp.minimum(lanes + 1, 15)
    last = lanes == 15
    neg = jnp.full((16,), NEG, F32)

    def init_body(i, carry):
        for f in range(FPT):
            tables[f][pl.ds(i * 16, 16)] = neg
        return carry
    lax.fori_loop(0, NCELL // 16, init_body, 0)

    def group_body(g, carry):
        base = g * 16
        c = idx_v[pl.ds(base, 16)]
        sk, p = plsc.sort_key_val(c, lanes)
        masks = [(sk == _vshuf(sk, si)) & ge
                 for si, ge in zip(shift_idx, ge_masks)]
        endm = (sk != _vshuf(sk, succ)) | last
        for f in range(FPT):
            v = feat_v[f, pl.ds(base, 16)]
            vs = _vshuf(v, p)
            for m, si in zip(masks, shift_idx):
                vs = jnp.where(m, jnp.maximum(vs, _vshuf(vs, si)), vs)
            old = plsc.load_gather(tables[f], [sk])
            plsc.store_scatter(tables[f], [sk], jnp.maximum(old, vs),
                               mask=endm)
        return carry
    lax.fori_loop(0, T // 16, group_body, 0)

    def back_body(g, carry):
        base = g * 16
        c = idx_v[pl.ds(base, 16)]
        for f in range(FPT):
            feat_v[f, pl.ds(base, 16)] = plsc.load_gather(tables[f], [c])
        return carry
    lax.fori_loop(0, T // 16, back_body, 0)

    pltpu.sync_copy(feat_v, out_hbm.at[b, pl.ds(f0, FPT), :])


_sc_pool = pl.kernel(
    _sc_pool_body,
    out_type=jax.ShapeDtypeStruct((B, HID, T), F32),
    mesh=plsc.VectorSubcoreMesh(core_axis_name="c", subcore_axis_name="s"),
    compiler_params=pltpu.CompilerParams(needs_layout_passes=False),
    scratch_types=(
        [pltpu.VMEM((T,), jnp.int32), pltpu.VMEM((FPT, T), F32)]
        + [pltpu.VMEM((NCELL,), F32) for _ in range(FPT)]
    ),
)


def _block_math(netT, poolT, w0aT, w0bT, b0, w1T, b1, wscaT, wscbT):
    x = netT[0]
    p = poolT[0]
    rx = jnp.maximum(x, 0.0)
    rp = jnp.maximum(p, 0.0)
    h = (jnp.dot(w0aT[...], rx, preferred_element_type=F32)
         + jnp.dot(w0bT[...], rp, preferred_element_type=F32) + b0[...])
    h = jnp.maximum(h, 0.0)
    dx = jnp.dot(w1T[...], h, preferred_element_type=F32) + b1[...]
    return (jnp.dot(wscaT[...], x, preferred_element_type=F32)
            + jnp.dot(wscbT[...], p, preferred_element_type=F32) + dx)


def _block_body(netT, poolT, w0aT, w0bT, b0, w1T, b1, wscaT, wscbT, out):
    out[0] = _block_math(netT, poolT, w0aT, w0bT, b0, w1T, b1, wscaT, wscbT)


def _final_body(netT, poolT, w0aT, w0bT, b0, w1T, b1, wscaT, wscbT,
                fcw, fcb, c_out):
    o = _block_math(netT, poolT, w0aT, w0bT, b0, w1T, b1, wscaT, wscbT)
    c_out[0] = lax.dot_general(o, fcw[...], (((0,), (0,)), ((), ())),
                               preferred_element_type=F32) + fcb[...]


def _wspec(a):
    return pl.BlockSpec(a.shape, lambda b: tuple(0 for _ in a.shape))


def _run_block(netT, poolT, wts):
    specs = ([pl.BlockSpec((1, HID, T), lambda b: (b, 0, 0))] * 2
             + [_wspec(w) for w in wts])
    return pl.pallas_call(
        _block_body,
        grid=(B,),
        in_specs=specs,
        out_specs=pl.BlockSpec((1, HID, T), lambda b: (b, 0, 0)),
        out_shape=jax.ShapeDtypeStruct((B, HID, T), F32),
    )(netT, poolT, *wts)


def _run_final(netT, poolT, wts):
    specs = ([pl.BlockSpec((1, HID, T), lambda b: (b, 0, 0))] * 2
             + [_wspec(w) for w in wts])
    return pl.pallas_call(
        _final_body,
        grid=(B,),
        in_specs=specs,
        out_specs=pl.BlockSpec((1, T, 64), lambda b: (b, 0, 0)),
        out_shape=jax.ShapeDtypeStruct((B, T, 64), F32),
    )(netT, poolT, *wts)


def kernel(p_start, p_end, state_start, state_end, state_target, params):
    prm = params
    psT = jnp.swapaxes(p_start, 1, 2)
    peT = jnp.swapaxes(p_end, 1, 2)
    states = jnp.stack([state_start, state_end, state_target], axis=1)

    sw1 = jnp.stack([prm['lin1'][0], prm['lin4'][0], prm['lin7'][0]])
    sb1 = jnp.stack([prm['lin1'][1], prm['lin4'][1], prm['lin7'][1]])[:, None, :]
    sw2 = jnp.stack([prm['lin2'][0], prm['lin5'][0], prm['lin8'][0]])
    sb2 = jnp.stack([prm['lin2'][1], prm['lin5'][1], prm['lin8'][1]])[:, None, :]
    sw3 = jnp.stack([prm['lin3'][0], prm['lin6'][0], prm['lin9'][0]])
    sb3 = jnp.stack([prm['lin3'][1], prm['lin6'][1], prm['lin9'][1]])[:, None, :]

    wsT = prm['Ws'][0].T
    bs = prm['Ws'][1][:, None]
    weT = prm['We'][0].T
    be = prm['We'][1][:, None]

    w0, b0, w1, b1, wsc = prm['blocks'][0]
    blk0T = (w0.T, b0[:, None], w1.T, b1[:, None], wsc.T)

    netT, idx3, state_feat = _run_attn(
        states, psT, peT, (sw1, sb1, sw2, sb2, sw3, sb3),
        wsT, bs, weT, be, blk0T)

    def blk_wts(blk):
        w0, b0, w1, b1, wsc = blk
        return (w0[:HID].T, w0[HID:].T, b0[:, None], w1.T, b1[:, None],
                wsc[:HID].T, wsc[HID:].T)

    for blk in prm['blocks'][1:-1]:
        poolT = _sc_pool(idx3, netT)
        netT = _run_block(netT, poolT, blk_wts(blk))

    poolT = _sc_pool(idx3, netT)
    fcw, fcb = prm['fc_c']
    c = _run_final(netT, poolT, blk_wts(prm['blocks'][-1]) + (fcw, fcb[None, :]))
    return (c, state_feat)


# speedup vs baseline: 4.0563x; 1.2518x over previous
import functools

import jax
import jax.numpy as jnp
from jax import lax
from jax.experimental import pallas as pl
from jax.experimental.pallas import tpu as pltpu
from jax.experimental.pallas import tpu_sc as plsc

B = 4
T = 2048
HID = 128
TWOH = 256
RESO = 64
NCELL = RESO * RESO
NT = 4
TT = T // NT
FPT = 16
NEG = -3.0e38
F32 = jnp.float32


def _attn_body(states, psT, peT, sw1, sb1, sw2, sb2, sw3, sb3,
               wsT, bs, weT, be, w0T, b0, w1T, b1, wscT,
               netT_out, idx_out, sf_out):
    sv = states[...]
    outs = []
    for i in range(3):
        s = sv[:, i:i + 1]
        h = jnp.maximum(s * sw1[i] + sb1[i], 0.0)
        h = jnp.maximum(
            jnp.dot(h, sw2[i], preferred_element_type=F32) + sb2[i], 0.0)
        outs.append(jnp.dot(h, sw3[i], preferred_element_type=F32) + sb3[i])
    ss, se, st = outs
    st = (ss + st) * 0.5
    sf_out[...] = jnp.concatenate([ss, se, st], axis=1)

    ps = psT[0]
    pe = peT[0]
    fsT = jnp.dot(wsT[...], ps, preferred_element_type=F32) + bs[...]
    feT = jnp.dot(weT[...], pe, preferred_element_type=F32) + be[...]
    fsT = fsT * (1.0 / 16.0)
    scores = lax.dot_general(fsT, feT, (((0,), (0,)), ((), ())),
                             preferred_element_type=F32)
    m = jnp.max(scores, axis=1, keepdims=True)
    e = jnp.exp(scores - m)
    attn = e / jnp.sum(e, axis=1, keepdims=True)
    n0T = lax.dot_general(feT, attn, (((1,), (1,)), ((), ())),
                          preferred_element_type=F32)

    r = jnp.maximum(n0T, 0.0)
    h = jnp.maximum(jnp.dot(w0T[...], r, preferred_element_type=F32) + b0[...], 0.0)
    dx = jnp.dot(w1T[...], h, preferred_element_type=F32) + b1[...]
    netT_out[0] = jnp.dot(wscT[...], n0T, preferred_element_type=F32) + dx

    u0 = jnp.clip(ps[0:1, :] / 1.001 + 0.5, 0.0, 1.0 - 1e-6)
    u2 = jnp.clip(ps[2:3, :] / 1.001 + 0.5, 0.0, 1.0 - 1e-6)
    xi = (u0 * RESO).astype(jnp.int32)
    zi = (u2 * RESO).astype(jnp.int32)
    idx_out[0] = xi + RESO * zi


def _run_attn(states, psT, peT, sws, wsT, bs, weT, be, blk0T):
    w0T, b0, w1T, b1, wscT = blk0T
    sw1, sb1, sw2, sb2, sw3, sb3 = sws
    cst = lambda *dims: (lambda b, j: tuple(0 for _ in dims))
    return pl.pallas_call(
        _attn_body,
        grid=(B, NT),
        in_specs=[
            pl.BlockSpec((B, 3), lambda b, j: (0, 0)),
            pl.BlockSpec((1, 4, TT), lambda b, j: (b, 0, j)),
            pl.BlockSpec((1, 4, T), lambda b, j: (b, 0, 0)),
            pl.BlockSpec(sw1.shape, lambda b, j: (0, 0, 0)),
            pl.BlockSpec(sb1.shape, lambda b, j: (0, 0, 0)),
            pl.BlockSpec(sw2.shape, lambda b, j: (0, 0, 0)),
            pl.BlockSpec(sb2.shape, lambda b, j: (0, 0, 0)),
            pl.BlockSpec(sw3.shape, lambda b, j: (0, 0, 0)),
            pl.BlockSpec(sb3.shape, lambda b, j: (0, 0, 0)),
            pl.BlockSpec(wsT.shape, lambda b, j: (0, 0)),
            pl.BlockSpec(bs.shape, lambda b, j: (0, 0)),
            pl.BlockSpec(weT.shape, lambda b, j: (0, 0)),
            pl.BlockSpec(be.shape, lambda b, j: (0, 0)),
            pl.BlockSpec(w0T.shape, lambda b, j: (0, 0)),
            pl.BlockSpec(b0.shape, lambda b, j: (0, 0)),
            pl.BlockSpec(w1T.shape, lambda b, j: (0, 0)),
            pl.BlockSpec(b1.shape, lambda b, j: (0, 0)),
            pl.BlockSpec(wscT.shape, lambda b, j: (0, 0)),
        ],
        out_specs=[
            pl.BlockSpec((1, HID, TT), lambda b, j: (b, 0, j)),
            pl.BlockSpec((1, 1, TT), lambda b, j: (b, 0, j)),
            pl.BlockSpec((B, 3 * 256), lambda b, j: (0, 0)),
        ],
        out_shape=[
            jax.ShapeDtypeStruct((B, HID, T), F32),
            jax.ShapeDtypeStruct((B, 1, T), jnp.int32),
            jax.ShapeDtypeStruct((B, 3 * 256), F32),
        ],
    )(states, psT, peT, sw1, sb1, sw2, sb2, sw3, sb3,
      wsT, bs, weT, be, w0T, b0, w1T, b1, wscT)


def _vshuf(v, i):
    return jnp.take_along_axis(v, i, axis=0)


def _sc_pool_body(idx_hbm, netT_hbm, out_hbm, idx_v, feat_v, *tables):
    cid = lax.axis_index("c")
    sid = lax.axis_index("s")
    wid = sid * 2 + cid
    b = wid // 8
    f0 = (wid % 8) * FPT

    pltpu.sync_copy(idx_hbm.at[b, 0], idx_v)
    pltpu.sync_copy(netT_hbm.at[b, pl.ds(f0, FPT), :], feat_v)

    lanes = lax.iota(jnp.int32, 16)
    shift_idx = [jnp.maximum(lanes - s, 0) for s in (1, 2, 4, 8)]
    ge_masks = [lanes >= s for s in (1, 2, 4, 8)]
    succ = jnp.minimum(lanes + 1, 15)
    last = lanes == 15
    neg = jnp.full((16,), NEG, F32)

    def init_body(i, carry):
        for f in range(FPT):
            tables[f][pl.ds(i * 16, 16)] = neg
        return carry
    lax.fori_loop(0, NCELL // 16, init_body, 0)

    def group_body(g, carry):
        base = g * 16
        c = idx_v[pl.ds(base, 16)]
        sk, p = plsc.sort_key_val(c, lanes)
        masks = [(sk == _vshuf(sk, si)) & ge
                 for si, ge in zip(shift_idx, ge_masks)]
        endm = (sk != _vshuf(sk, succ)) | last
        vs = [_vshuf(feat_v[f, pl.ds(base, 16)], p) for f in range(FPT)]
        for m, si in zip(masks, shift_idx):
            sh = [_vshuf(v, si) for v in vs]
            vs = [jnp.where(m, jnp.maximum(v, s), v)
                  for v, s in zip(vs, sh)]
        olds = [plsc.load_gather(tables[f], [sk]) for f in range(FPT)]
        for f in range(FPT):
            plsc.store_scatter(tables[f], [sk],
                               jnp.maximum(olds[f], vs[f]), mask=endm)
        return carry
    lax.fori_loop(0, T // 16, group_body, 0)

    def back_body(g, carry):
        base = g * 16
        c = idx_v[pl.ds(base, 16)]
        for f in range(FPT):
            feat_v[f, pl.ds(base, 16)] = plsc.load_gather(tables[f], [c])
        return carry
    lax.fori_loop(0, T // 16, back_body, 0)

    pltpu.sync_copy(feat_v, out_hbm.at[b, pl.ds(f0, FPT), :])


_sc_pool = pl.kernel(
    _sc_pool_body,
    out_type=jax.ShapeDtypeStruct((B, HID, T), F32),
    mesh=plsc.VectorSubcoreMesh(core_axis_name="c", subcore_axis_name="s"),
    compiler_params=pltpu.CompilerParams(needs_layout_passes=False),
    scratch_types=(
        [pltpu.VMEM((T,), jnp.int32), pltpu.VMEM((FPT, T), F32)]
        + [pltpu.VMEM((NCELL,), F32) for _ in range(FPT)]
    ),
)


def _block_math(netT, poolT, w0aT, w0bT, b0, w1T, b1, wscaT, wscbT):
    x = netT[0]
    p = poolT[0]
    rx = jnp.maximum(x, 0.0)
    rp = jnp.maximum(p, 0.0)
    h = (jnp.dot(w0aT[...], rx, preferred_element_type=F32)
         + jnp.dot(w0bT[...], rp, preferred_element_type=F32) + b0[...])
    h = jnp.maximum(h, 0.0)
    dx = jnp.dot(w1T[...], h, preferred_element_type=F32) + b1[...]
    return (jnp.dot(wscaT[...], x, preferred_element_type=F32)
            + jnp.dot(wscbT[...], p, preferred_element_type=F32) + dx)


def _block_body(netT, poolT, w0aT, w0bT, b0, w1T, b1, wscaT, wscbT, out):
    out[0] = _block_math(netT, poolT, w0aT, w0bT, b0, w1T, b1, wscaT, wscbT)


def _final_body(netT, poolT, w0aT, w0bT, b0, w1T, b1, wscaT, wscbT,
                fcw, fcb, c_out):
    o = _block_math(netT, poolT, w0aT, w0bT, b0, w1T, b1, wscaT, wscbT)
    c_out[0] = lax.dot_general(o, fcw[...], (((0,), (0,)), ((), ())),
                               preferred_element_type=F32) + fcb[...]


def _wspec(a):
    return pl.BlockSpec(a.shape, lambda b: tuple(0 for _ in a.shape))


def _run_block(netT, poolT, wts):
    specs = ([pl.BlockSpec((1, HID, T), lambda b: (b, 0, 0))] * 2
             + [_wspec(w) for w in wts])
    return pl.pallas_call(
        _block_body,
        grid=(B,),
        in_specs=specs,
        out_specs=pl.BlockSpec((1, HID, T), lambda b: (b, 0, 0)),
        out_shape=jax.ShapeDtypeStruct((B, HID, T), F32),
    )(netT, poolT, *wts)


def _run_final(netT, poolT, wts):
    specs = ([pl.BlockSpec((1, HID, T), lambda b: (b, 0, 0))] * 2
             + [_wspec(w) for w in wts])
    return pl.pallas_call(
        _final_body,
        grid=(B,),
        in_specs=specs,
        out_specs=pl.BlockSpec((1, T, 64), lambda b: (b, 0, 0)),
        out_shape=jax.ShapeDtypeStruct((B, T, 64), F32),
    )(netT, poolT, *wts)


def kernel(p_start, p_end, state_start, state_end, state_target, params):
    prm = params
    psT = jnp.swapaxes(p_start, 1, 2)
    peT = jnp.swapaxes(p_end, 1, 2)
    states = jnp.stack([state_start, state_end, state_target], axis=1)

    sw1 = jnp.stack([prm['lin1'][0], prm['lin4'][0], prm['lin7'][0]])
    sb1 = jnp.stack([prm['lin1'][1], prm['lin4'][1], prm['lin7'][1]])[:, None, :]
    sw2 = jnp.stack([prm['lin2'][0], prm['lin5'][0], prm['lin8'][0]])
    sb2 = jnp.stack([prm['lin2'][1], prm['lin5'][1], prm['lin8'][1]])[:, None, :]
    sw3 = jnp.stack([prm['lin3'][0], prm['lin6'][0], prm['lin9'][0]])
    sb3 = jnp.stack([prm['lin3'][1], prm['lin6'][1], prm['lin9'][1]])[:, None, :]

    wsT = prm['Ws'][0].T
    bs = prm['Ws'][1][:, None]
    weT = prm['We'][0].T
    be = prm['We'][1][:, None]

    w0, b0, w1, b1, wsc = prm['blocks'][0]
    blk0T = (w0.T, b0[:, None], w1.T, b1[:, None], wsc.T)

    netT, idx3, state_feat = _run_attn(
        states, psT, peT, (sw1, sb1, sw2, sb2, sw3, sb3),
        wsT, bs, weT, be, blk0T)

    def blk_wts(blk):
        w0, b0, w1, b1, wsc = blk
        return (w0[:HID].T, w0[HID:].T, b0[:, None], w1.T, b1[:, None],
                wsc[:HID].T, wsc[HID:].T)

    for blk in prm['blocks'][1:-1]:
        poolT = _sc_pool(idx3, netT)
        netT = _run_block(netT, poolT, blk_wts(blk))

    poolT = _sc_pool(idx3, netT)
    fcw, fcb = prm['fc_c']
    c = _run_final(netT, poolT, blk_wts(prm['blocks'][-1]) + (fcw, fcb[None, :]))
    return (c, state_feat)


# speedup vs baseline: 4.1319x; 1.0186x over previous
import functools

import jax
import jax.numpy as jnp
from jax import lax
from jax.experimental import pallas as pl
from jax.experimental.pallas import tpu as pltpu
from jax.experimental.pallas import tpu_sc as plsc

B = 4
T = 2048
HID = 128
TWOH = 256
RESO = 64
NCELL = RESO * RESO
NT = 4
TT = T // NT
FPT = 16
NEG = -3.0e38
F32 = jnp.float32


def _attn_body(states, psT, peT, sw1, sb1, sw2, sb2, sw3, sb3,
               wsT, bs, weT, be, w0T, b0, w1T, b1, wscT,
               netT_out, idx_out, sf_out):
    sv = states[...]
    outs = []
    for i in range(3):
        s = sv[:, i:i + 1]
        h = jnp.maximum(s * sw1[i] + sb1[i], 0.0)
        h = jnp.maximum(
            jnp.dot(h, sw2[i], preferred_element_type=F32) + sb2[i], 0.0)
        outs.append(jnp.dot(h, sw3[i], preferred_element_type=F32) + sb3[i])
    ss, se, st = outs
    st = (ss + st) * 0.5
    sf_out[...] = jnp.concatenate([ss, se, st], axis=1)

    ps = psT[0]
    pe = peT[0]
    fsT = jnp.dot(wsT[...], ps, preferred_element_type=F32) + bs[...]
    feT = jnp.dot(weT[...], pe, preferred_element_type=F32) + be[...]
    fsT = fsT * (1.0 / 16.0)
    scores = lax.dot_general(fsT, feT, (((0,), (0,)), ((), ())),
                             preferred_element_type=F32)
    m = jnp.max(scores, axis=1, keepdims=True)
    e = jnp.exp(scores - m)
    attn = e / jnp.sum(e, axis=1, keepdims=True)
    n0T = lax.dot_general(feT, attn, (((1,), (1,)), ((), ())),
                          preferred_element_type=F32)

    r = jnp.maximum(n0T, 0.0)
    h = jnp.maximum(jnp.dot(w0T[...], r, preferred_element_type=F32) + b0[...], 0.0)
    dx = jnp.dot(w1T[...], h, preferred_element_type=F32) + b1[...]
    netT_out[0] = jnp.dot(wscT[...], n0T, preferred_element_type=F32) + dx

    u0 = jnp.clip(ps[0:1, :] / 1.001 + 0.5, 0.0, 1.0 - 1e-6)
    u2 = jnp.clip(ps[2:3, :] / 1.001 + 0.5, 0.0, 1.0 - 1e-6)
    xi = (u0 * RESO).astype(jnp.int32)
    zi = (u2 * RESO).astype(jnp.int32)
    idx_out[0] = xi + RESO * zi


def _run_attn(states, psT, peT, sws, wsT, bs, weT, be, blk0T):
    w0T, b0, w1T, b1, wscT = blk0T
    sw1, sb1, sw2, sb2, sw3, sb3 = sws
    cst = lambda *dims: (lambda b, j: tuple(0 for _ in dims))
    return pl.pallas_call(
        _attn_body,
        grid=(B, NT),
        in_specs=[
            pl.BlockSpec((B, 3), lambda b, j: (0, 0)),
            pl.BlockSpec((1, 4, TT), lambda b, j: (b, 0, j)),
            pl.BlockSpec((1, 4, T), lambda b, j: (b, 0, 0)),
            pl.BlockSpec(sw1.shape, lambda b, j: (0, 0, 0)),
            pl.BlockSpec(sb1.shape, lambda b, j: (0, 0, 0)),
            pl.BlockSpec(sw2.shape, lambda b, j: (0, 0, 0)),
            pl.BlockSpec(sb2.shape, lambda b, j: (0, 0, 0)),
            pl.BlockSpec(sw3.shape, lambda b, j: (0, 0, 0)),
            pl.BlockSpec(sb3.shape, lambda b, j: (0, 0, 0)),
            pl.BlockSpec(wsT.shape, lambda b, j: (0, 0)),
            pl.BlockSpec(bs.shape, lambda b, j: (0, 0)),
            pl.BlockSpec(weT.shape, lambda b, j: (0, 0)),
            pl.BlockSpec(be.shape, lambda b, j: (0, 0)),
            pl.BlockSpec(w0T.shape, lambda b, j: (0, 0)),
            pl.BlockSpec(b0.shape, lambda b, j: (0, 0)),
            pl.BlockSpec(w1T.shape, lambda b, j: (0, 0)),
            pl.BlockSpec(b1.shape, lambda b, j: (0, 0)),
            pl.BlockSpec(wscT.shape, lambda b, j: (0, 0)),
        ],
        out_specs=[
            pl.BlockSpec((1, HID, TT), lambda b, j: (b, 0, j)),
            pl.BlockSpec((1, 1, TT), lambda b, j: (b, 0, j)),
            pl.BlockSpec((B, 3 * 256), lambda b, j: (0, 0)),
        ],
        out_shape=[
            jax.ShapeDtypeStruct((B, HID, T), F32),
            jax.ShapeDtypeStruct((B, 1, T), jnp.int32),
            jax.ShapeDtypeStruct((B, 3 * 256), F32),
        ],
    )(states, psT, peT, sw1, sb1, sw2, sb2, sw3, sb3,
      wsT, bs, weT, be, w0T, b0, w1T, b1, wscT)


def _vshuf(v, i):
    return jnp.take_along_axis(v, i, axis=0)


def _sc_pool_body(idx_hbm, netT_hbm, out_hbm, idx_v, feat_v, sem1, sem2,
                  *tables):
    cid = lax.axis_index("c")
    sid = lax.axis_index("s")
    wid = sid * 2 + cid
    b = wid // 8
    f0 = (wid % 8) * FPT

    cp1 = pltpu.async_copy(idx_hbm.at[b, 0], idx_v, sem1)
    cp2 = pltpu.async_copy(netT_hbm.at[b, pl.ds(f0, FPT), :], feat_v, sem2)

    lanes = lax.iota(jnp.int32, 16)
    shift_idx = [jnp.maximum(lanes - s, 0) for s in (1, 2, 4, 8)]
    ge_masks = [lanes >= s for s in (1, 2, 4, 8)]
    succ = jnp.minimum(lanes + 1, 15)
    last = lanes == 15
    neg = jnp.full((16,), NEG, F32)

    def init_body(i, carry):
        for f in range(FPT):
            tables[f][pl.ds(i * 16, 16)] = neg
        return carry
    lax.fori_loop(0, NCELL // 16, init_body, 0)
    cp1.wait()
    cp2.wait()

    def stage(vs, m, si):
        return tuple(jnp.where(m, jnp.maximum(v, _vshuf(v, si)), v)
                     for v in vs)

    def group_body(g, carry):
        base = g * 16
        c = idx_v[pl.ds(base, 16)]
        sk, p = plsc.sort_key_val(c, lanes)
        masks = [(sk == _vshuf(sk, si)) & ge
                 for si, ge in zip(shift_idx, ge_masks)]
        endm = (sk != _vshuf(sk, succ)) | last
        vs = tuple(_vshuf(feat_v[f, pl.ds(base, 16)], p) for f in range(FPT))
        vs = stage(vs, masks[0], shift_idx[0])

        def deep(vs):
            vs = stage(vs, masks[2], shift_idx[2])
            return lax.cond(jnp.any(masks[3]),
                            lambda a: stage(a, masks[3], shift_idx[3]),
                            lambda a: a, vs)

        vs = lax.cond(jnp.any(masks[1]),
                      lambda a: stage(a, masks[1], shift_idx[1]),
                      lambda a: a, vs)
        vs = lax.cond(jnp.any(masks[2]), deep, lambda a: a, vs)
        olds = [plsc.load_gather(tables[f], [sk]) for f in range(FPT)]
        for f in range(FPT):
            plsc.store_scatter(tables[f], [sk],
                               jnp.maximum(olds[f], vs[f]), mask=endm)
        return carry
    lax.fori_loop(0, T // 16, group_body, 0)

    def back_body(g, carry):
        base = g * 16
        c = idx_v[pl.ds(base, 16)]
        for f in range(FPT):
            feat_v[f, pl.ds(base, 16)] = plsc.load_gather(tables[f], [c])
        return carry
    lax.fori_loop(0, T // 16, back_body, 0)

    pltpu.sync_copy(feat_v, out_hbm.at[b, pl.ds(f0, FPT), :])


_sc_pool = pl.kernel(
    _sc_pool_body,
    out_type=jax.ShapeDtypeStruct((B, HID, T), F32),
    mesh=plsc.VectorSubcoreMesh(core_axis_name="c", subcore_axis_name="s"),
    compiler_params=pltpu.CompilerParams(needs_layout_passes=False),
    scratch_types=(
        [pltpu.VMEM((T,), jnp.int32), pltpu.VMEM((FPT, T), F32),
         pltpu.SemaphoreType.DMA, pltpu.SemaphoreType.DMA]
        + [pltpu.VMEM((NCELL,), F32) for _ in range(FPT)]
    ),
)


def _block_math(netT, poolT, w0aT, w0bT, b0, w1T, b1, wscaT, wscbT):
    x = netT[0]
    p = poolT[0]
    rx = jnp.maximum(x, 0.0)
    rp = jnp.maximum(p, 0.0)
    h = (jnp.dot(w0aT[...], rx, preferred_element_type=F32)
         + jnp.dot(w0bT[...], rp, preferred_element_type=F32) + b0[...])
    h = jnp.maximum(h, 0.0)
    dx = jnp.dot(w1T[...], h, preferred_element_type=F32) + b1[...]
    return (jnp.dot(wscaT[...], x, preferred_element_type=F32)
            + jnp.dot(wscbT[...], p, preferred_element_type=F32) + dx)


def _block_body(netT, poolT, w0aT, w0bT, b0, w1T, b1, wscaT, wscbT, out):
    out[0] = _block_math(netT, poolT, w0aT, w0bT, b0, w1T, b1, wscaT, wscbT)


def _final_body(netT, poolT, w0aT, w0bT, b0, w1T, b1, wscaT, wscbT,
                fcw, fcb, c_out):
    o = _block_math(netT, poolT, w0aT, w0bT, b0, w1T, b1, wscaT, wscbT)
    c_out[0] = lax.dot_general(o, fcw[...], (((0,), (0,)), ((), ())),
                               preferred_element_type=F32) + fcb[...]


def _wspec(a):
    return pl.BlockSpec(a.shape, lambda b: tuple(0 for _ in a.shape))


def _run_block(netT, poolT, wts):
    specs = ([pl.BlockSpec((1, HID, T), lambda b: (b, 0, 0))] * 2
             + [_wspec(w) for w in wts])
    return pl.pallas_call(
        _block_body,
        grid=(B,),
        in_specs=specs,
        out_specs=pl.BlockSpec((1, HID, T), lambda b: (b, 0, 0)),
        out_shape=jax.ShapeDtypeStruct((B, HID, T), F32),
    )(netT, poolT, *wts)


def _run_final(netT, poolT, wts):
    specs = ([pl.BlockSpec((1, HID, T), lambda b: (b, 0, 0))] * 2
             + [_wspec(w) for w in wts])
    return pl.pallas_call(
        _final_body,
        grid=(B,),
        in_specs=specs,
        out_specs=pl.BlockSpec((1, T, 64), lambda b: (b, 0, 0)),
        out_shape=jax.ShapeDtypeStruct((B, T, 64), F32),
    )(netT, poolT, *wts)


def kernel(p_start, p_end, state_start, state_end, state_target, params):
    prm = params
    psT = jnp.swapaxes(p_start, 1, 2)
    peT = jnp.swapaxes(p_end, 1, 2)
    states = jnp.stack([state_start, state_end, state_target], axis=1)

    sw1 = jnp.stack([prm['lin1'][0], prm['lin4'][0], prm['lin7'][0]])
    sb1 = jnp.stack([prm['lin1'][1], prm['lin4'][1], prm['lin7'][1]])[:, None, :]
    sw2 = jnp.stack([prm['lin2'][0], prm['lin5'][0], prm['lin8'][0]])
    sb2 = jnp.stack([prm['lin2'][1], prm['lin5'][1], prm['lin8'][1]])[:, None, :]
    sw3 = jnp.stack([prm['lin3'][0], prm['lin6'][0], prm['lin9'][0]])
    sb3 = jnp.stack([prm['lin3'][1], prm['lin6'][1], prm['lin9'][1]])[:, None, :]

    wsT = prm['Ws'][0].T
    bs = prm['Ws'][1][:, None]
    weT = prm['We'][0].T
    be = prm['We'][1][:, None]

    w0, b0, w1, b1, wsc = prm['blocks'][0]
    blk0T = (w0.T, b0[:, None], w1.T, b1[:, None], wsc.T)

    netT, idx3, state_feat = _run_attn(
        states, psT, peT, (sw1, sb1, sw2, sb2, sw3, sb3),
        wsT, bs, weT, be, blk0T)

    def blk_wts(blk):
        w0, b0, w1, b1, wsc = blk
        return (w0[:HID].T, w0[HID:].T, b0[:, None], w1.T, b1[:, None],
                wsc[:HID].T, wsc[HID:].T)

    for blk in prm['blocks'][1:-1]:
        poolT = _sc_pool(idx3, netT)
        netT = _run_block(netT, poolT, blk_wts(blk))

    poolT = _sc_pool(idx3, netT)
    fcw, fcb = prm['fc_c']
    c = _run_final(netT, poolT, blk_wts(prm['blocks'][-1]) + (fcw, fcb[None, :]))
    return (c, state_feat)


# speedup vs baseline: 4.2415x; 1.0265x over previous
import functools

import jax
import jax.numpy as jnp
from jax import lax
from jax.experimental import pallas as pl
from jax.experimental.pallas import tpu as pltpu
from jax.experimental.pallas import tpu_sc as plsc

B = 4
T = 2048
HID = 128
TWOH = 256
RESO = 64
NCELL = RESO * RESO
NT = 4
TT = T // NT
FPT = 16
NEG = -3.0e38
F32 = jnp.float32


def _attn_body(states, psT, peT, sw1, sb1, sw2, sb2, sw3, sb3,
               wsT, bs, weT, be, w0T, b0, w1T, b1, wscT,
               netT_out, idx_out, sf_out):
    sv = states[...]
    outs = []
    for i in range(3):
        s = sv[:, i:i + 1]
        h = jnp.maximum(s * sw1[i] + sb1[i], 0.0)
        h = jnp.maximum(
            jnp.dot(h, sw2[i], preferred_element_type=F32) + sb2[i], 0.0)
        outs.append(jnp.dot(h, sw3[i], preferred_element_type=F32) + sb3[i])
    ss, se, st = outs
    st = (ss + st) * 0.5
    sf_out[...] = jnp.concatenate([ss, se, st], axis=1)

    ps = psT[0]
    pe = peT[0]
    fsT = jnp.dot(wsT[...], ps, preferred_element_type=F32) + bs[...]
    feT = jnp.dot(weT[...], pe, preferred_element_type=F32) + be[...]
    fsT = fsT * (1.0 / 16.0)
    scores = lax.dot_general(fsT, feT, (((0,), (0,)), ((), ())),
                             preferred_element_type=F32)
    m = jnp.max(scores, axis=1, keepdims=True)
    e = jnp.exp(scores - m)
    attn = e / jnp.sum(e, axis=1, keepdims=True)
    n0T = lax.dot_general(feT, attn, (((1,), (1,)), ((), ())),
                          preferred_element_type=F32)

    r = jnp.maximum(n0T, 0.0)
    h = jnp.maximum(jnp.dot(w0T[...], r, preferred_element_type=F32) + b0[...], 0.0)
    dx = jnp.dot(w1T[...], h, preferred_element_type=F32) + b1[...]
    netT_out[0] = jnp.dot(wscT[...], n0T, preferred_element_type=F32) + dx

    u0 = jnp.clip(ps[0:1, :] / 1.001 + 0.5, 0.0, 1.0 - 1e-6)
    u2 = jnp.clip(ps[2:3, :] / 1.001 + 0.5, 0.0, 1.0 - 1e-6)
    xi = (u0 * RESO).astype(jnp.int32)
    zi = (u2 * RESO).astype(jnp.int32)
    idx_out[0] = xi + RESO * zi


def _run_attn(states, psT, peT, sws, wsT, bs, weT, be, blk0T):
    w0T, b0, w1T, b1, wscT = blk0T
    sw1, sb1, sw2, sb2, sw3, sb3 = sws
    cst = lambda *dims: (lambda b, j: tuple(0 for _ in dims))
    return pl.pallas_call(
        _attn_body,
        grid=(B, NT),
        in_specs=[
            pl.BlockSpec((B, 3), lambda b, j: (0, 0)),
            pl.BlockSpec((1, 4, TT), lambda b, j: (b, 0, j)),
            pl.BlockSpec((1, 4, T), lambda b, j: (b, 0, 0)),
            pl.BlockSpec(sw1.shape, lambda b, j: (0, 0, 0)),
            pl.BlockSpec(sb1.shape, lambda b, j: (0, 0, 0)),
            pl.BlockSpec(sw2.shape, lambda b, j: (0, 0, 0)),
            pl.BlockSpec(sb2.shape, lambda b, j: (0, 0, 0)),
            pl.BlockSpec(sw3.shape, lambda b, j: (0, 0, 0)),
            pl.BlockSpec(sb3.shape, lambda b, j: (0, 0, 0)),
            pl.BlockSpec(wsT.shape, lambda b, j: (0, 0)),
            pl.BlockSpec(bs.shape, lambda b, j: (0, 0)),
            pl.BlockSpec(weT.shape, lambda b, j: (0, 0)),
            pl.BlockSpec(be.shape, lambda b, j: (0, 0)),
            pl.BlockSpec(w0T.shape, lambda b, j: (0, 0)),
            pl.BlockSpec(b0.shape, lambda b, j: (0, 0)),
            pl.BlockSpec(w1T.shape, lambda b, j: (0, 0)),
            pl.BlockSpec(b1.shape, lambda b, j: (0, 0)),
            pl.BlockSpec(wscT.shape, lambda b, j: (0, 0)),
        ],
        out_specs=[
            pl.BlockSpec((1, HID, TT), lambda b, j: (b, 0, j)),
            pl.BlockSpec((1, 1, TT), lambda b, j: (b, 0, j)),
            pl.BlockSpec((B, 3 * 256), lambda b, j: (0, 0)),
        ],
        out_shape=[
            jax.ShapeDtypeStruct((B, HID, T), F32),
            jax.ShapeDtypeStruct((B, 1, T), jnp.int32),
            jax.ShapeDtypeStruct((B, 3 * 256), F32),
        ],
    )(states, psT, peT, sw1, sb1, sw2, sb2, sw3, sb3,
      wsT, bs, weT, be, w0T, b0, w1T, b1, wscT)


def _vshuf(v, i):
    return jnp.take_along_axis(v, i, axis=0)


def _sc_pool_body(idx_hbm, netT_hbm, out_hbm, idx_v, feat_v, sem1, sem2,
                  *tables):
    cid = lax.axis_index("c")
    sid = lax.axis_index("s")
    wid = sid * 2 + cid
    b = wid // 8
    f0 = (wid % 8) * FPT

    cp1 = pltpu.async_copy(idx_hbm.at[b, 0], idx_v, sem1)
    cp2 = pltpu.async_copy(netT_hbm.at[b, pl.ds(f0, FPT), :], feat_v, sem2)

    lanes = lax.iota(jnp.int32, 16)
    shift_idx = [jnp.maximum(lanes - s, 0) for s in (1, 2, 4, 8)]
    ge_masks = [lanes >= s for s in (1, 2, 4, 8)]
    succ = jnp.minimum(lanes + 1, 15)
    last = lanes == 15
    neg = jnp.full((16,), NEG, F32)

    def init_body(i, carry):
        for f in range(FPT):
            tables[f][pl.ds(i * 16, 16)] = neg
        return carry
    lax.fori_loop(0, NCELL // 16, init_body, 0)
    cp1.wait()
    cp2.wait()

    def stage(vs, m, si):
        return tuple(jnp.where(m, jnp.maximum(v, _vshuf(v, si)), v)
                     for v in vs)

    def group_body(g, carry):
        base = g * 16
        c = idx_v[pl.ds(base, 16)]
        sk, p = plsc.sort_key_val(c, lanes)
        masks = [(sk == _vshuf(sk, si)) & ge
                 for si, ge in zip(shift_idx, ge_masks)]
        endm = (sk != _vshuf(sk, succ)) | last
        vs = tuple(_vshuf(feat_v[f, pl.ds(base, 16)], p) for f in range(FPT))
        vs = stage(vs, masks[0], shift_idx[0])

        def deep(vs):
            vs = stage(vs, masks[2], shift_idx[2])
            return lax.cond(jnp.any(masks[3]),
                            lambda a: stage(a, masks[3], shift_idx[3]),
                            lambda a: a, vs)

        vs = lax.cond(jnp.any(masks[1]),
                      lambda a: stage(a, masks[1], shift_idx[1]),
                      lambda a: a, vs)
        vs = lax.cond(jnp.any(masks[2]), deep, lambda a: a, vs)
        olds = [plsc.load_gather(tables[f], [sk], mask=endm)
                for f in range(FPT)]
        for f in range(FPT):
            plsc.store_scatter(tables[f], [sk],
                               jnp.maximum(olds[f], vs[f]), mask=endm)
        return carry
    lax.fori_loop(0, T // 16, group_body, 0)

    def back_body(g, carry):
        base = g * 16
        c = idx_v[pl.ds(base, 16)]
        for f in range(FPT):
            feat_v[f, pl.ds(base, 16)] = plsc.load_gather(tables[f], [c])
        return carry
    lax.fori_loop(0, T // 16, back_body, 0)

    pltpu.sync_copy(feat_v, out_hbm.at[b, pl.ds(f0, FPT), :])


_sc_pool = pl.kernel(
    _sc_pool_body,
    out_type=jax.ShapeDtypeStruct((B, HID, T), F32),
    mesh=plsc.VectorSubcoreMesh(core_axis_name="c", subcore_axis_name="s"),
    compiler_params=pltpu.CompilerParams(needs_layout_passes=False),
    scratch_types=(
        [pltpu.VMEM((T,), jnp.int32), pltpu.VMEM((FPT, T), F32),
         pltpu.SemaphoreType.DMA, pltpu.SemaphoreType.DMA]
        + [pltpu.VMEM((NCELL,), F32) for _ in range(FPT)]
    ),
)


def _block_math(netT, poolT, w0aT, w0bT, b0, w1T, b1, wscaT, wscbT):
    x = netT[0]
    p = poolT[0]
    rx = jnp.maximum(x, 0.0)
    rp = jnp.maximum(p, 0.0)
    h = (jnp.dot(w0aT[...], rx, preferred_element_type=F32)
         + jnp.dot(w0bT[...], rp, preferred_element_type=F32) + b0[...])
    h = jnp.maximum(h, 0.0)
    dx = jnp.dot(w1T[...], h, preferred_element_type=F32) + b1[...]
    return (jnp.dot(wscaT[...], x, preferred_element_type=F32)
            + jnp.dot(wscbT[...], p, preferred_element_type=F32) + dx)


def _block_body(netT, poolT, w0aT, w0bT, b0, w1T, b1, wscaT, wscbT, out):
    out[0] = _block_math(netT, poolT, w0aT, w0bT, b0, w1T, b1, wscaT, wscbT)


def _final_body(netT, poolT, w0aT, w0bT, b0, w1T, b1, wscaT, wscbT,
                fcw, fcb, c_out):
    o = _block_math(netT, poolT, w0aT, w0bT, b0, w1T, b1, wscaT, wscbT)
    c_out[0] = lax.dot_general(o, fcw[...], (((0,), (0,)), ((), ())),
                               preferred_element_type=F32) + fcb[...]


def _wspec(a):
    return pl.BlockSpec(a.shape, lambda b: tuple(0 for _ in a.shape))


def _run_block(netT, poolT, wts):
    specs = ([pl.BlockSpec((1, HID, T), lambda b: (b, 0, 0))] * 2
             + [_wspec(w) for w in wts])
    return pl.pallas_call(
        _block_body,
        grid=(B,),
        in_specs=specs,
        out_specs=pl.BlockSpec((1, HID, T), lambda b: (b, 0, 0)),
        out_shape=jax.ShapeDtypeStruct((B, HID, T), F32),
    )(netT, poolT, *wts)


def _run_final(netT, poolT, wts):
    specs = ([pl.BlockSpec((1, HID, T), lambda b: (b, 0, 0))] * 2
             + [_wspec(w) for w in wts])
    return pl.pallas_call(
        _final_body,
        grid=(B,),
        in_specs=specs,
        out_specs=pl.BlockSpec((1, T, 64), lambda b: (b, 0, 0)),
        out_shape=jax.ShapeDtypeStruct((B, T, 64), F32),
    )(netT, poolT, *wts)


def kernel(p_start, p_end, state_start, state_end, state_target, params):
    prm = params
    psT = jnp.swapaxes(p_start, 1, 2)
    peT = jnp.swapaxes(p_end, 1, 2)
    states = jnp.stack([state_start, state_end, state_target], axis=1)

    sw1 = jnp.stack([prm['lin1'][0], prm['lin4'][0], prm['lin7'][0]])
    sb1 = jnp.stack([prm['lin1'][1], prm['lin4'][1], prm['lin7'][1]])[:, None, :]
    sw2 = jnp.stack([prm['lin2'][0], prm['lin5'][0], prm['lin8'][0]])
    sb2 = jnp.stack([prm['lin2'][1], prm['lin5'][1], prm['lin8'][1]])[:, None, :]
    sw3 = jnp.stack([prm['lin3'][0], prm['lin6'][0], prm['lin9'][0]])
    sb3 = jnp.stack([prm['lin3'][1], prm['lin6'][1], prm['lin9'][1]])[:, None, :]

    wsT = prm['Ws'][0].T
    bs = prm['Ws'][1][:, None]
    weT = prm['We'][0].T
    be = prm['We'][1][:, None]

    w0, b0, w1, b1, wsc = prm['blocks'][0]
    blk0T = (w0.T, b0[:, None], w1.T, b1[:, None], wsc.T)

    netT, idx3, state_feat = _run_attn(
        states, psT, peT, (sw1, sb1, sw2, sb2, sw3, sb3),
        wsT, bs, weT, be, blk0T)

    def blk_wts(blk):
        w0, b0, w1, b1, wsc = blk
        return (w0[:HID].T, w0[HID:].T, b0[:, None], w1.T, b1[:, None],
                wsc[:HID].T, wsc[HID:].T)

    for blk in prm['blocks'][1:-1]:
        poolT = _sc_pool(idx3, netT)
        netT = _run_block(netT, poolT, blk_wts(blk))

    poolT = _sc_pool(idx3, netT)
    fcw, fcb = prm['fc_c']
    c = _run_final(netT, poolT, blk_wts(prm['blocks'][-1]) + (fcw, fcb[None, :]))
    return (c, state_feat)
